# Initial kernel scaffold; baseline (speedup 1.0000x reference)
import jax
import jax.numpy as jnp
from jax.experimental import pallas as pl


def kernel(x, edge_index, Wl0, bl0, Wr0, Wl1, bl1, Wr1, Wl2, bl2, Wr2):
    raise NotImplementedError("write your pallas kernel here")


# speedup vs baseline: 1.5949x; 1.5949x over previous
import functools

import jax
import jax.numpy as jnp
from jax import lax
from jax.experimental import pallas as pl
from jax.experimental.pallas import tpu as pltpu
from jax.experimental.pallas import tpu_sc as plsc

N = 10000
E = 160000
D = 256
HALF = 128
NC = 2
NS = 16
NH = 2
NHN = N // NH
CHUNK = 128
CH_PER_TILE = 80
NT = NS * CH_PER_TILE
EP = NT * CHUNK
RPT = 320
RH = NS * RPT
DUMP = NHN
ZCH = -(-RPT // CHUNK)
IDXC = 8

BR = 1000
GRID = N // BR
BPH = NHN // BR

_mesh = plsc.VectorSubcoreMesh(core_axis_name="c", subcore_axis_name="s")


def _sc_body(with_deg, *refs):
    if with_deg:
        (y_hbm, src_hbm, dst_hbm, agg_hbm, deg_hbm,
         src_v, dst_v, buf_a, buf_b, acc_sh, sem_a, sem_b) = refs
    else:
        (y_hbm, src_hbm, dst_hbm, agg_hbm,
         src_v, dst_v, buf_a, buf_b, acc_sh, sem_a, sem_b) = refs

    c = lax.axis_index("c")
    s = lax.axis_index("s")
    rowbase = s * RPT

    def _fill(buf, val):
        def _row(r, _):
            for k in range(HALF // 16):
                buf[r, pl.ds(k * 16, 16)] = jnp.full((16,), val, jnp.float32)
            return 0
        lax.fori_loop(0, CHUNK, _row, 0)

    def _zero_acc():
        for j in range(ZCH):
            n = min(CHUNK, RPT - j * CHUNK)
            pltpu.sync_copy(buf_b.at[pl.ds(0, n)],
                            acc_sh.at[pl.ds(rowbase + j * CHUNK, n)])

    if with_deg:
        _fill(buf_a, 1.0)
        _fill(buf_b, 0.0)
        _zero_acc()
        plsc.subcore_barrier()

        def _deg_chunk(j, _):
            pltpu.sync_copy(buf_a, acc_sh.at[dst_v.at[j]], add=True)
            return 0

        for g in range(CH_PER_TILE // IDXC):
            base = s * CH_PER_TILE + g * IDXC
            pltpu.sync_copy(dst_hbm.at[c, pl.ds(base, IDXC)], dst_v)
            lax.fori_loop(0, IDXC, _deg_chunk, 0)

        plsc.subcore_barrier()
        pltpu.sync_copy(acc_sh.at[pl.ds(rowbase, RPT)],
                        deg_hbm.at[c, pl.ds(rowbase, RPT)])
        plsc.subcore_barrier()

    for p in range(NH):
        _fill(buf_b, 0.0)
        _zero_acc()
        plsc.subcore_barrier()

        y_c = y_hbm.at[c]

        def _pair(j2, _):
            j = j2 * 2
            cp_a = pltpu.async_copy(y_c.at[src_v.at[j]], buf_a, sem_a)
            cp_b = pltpu.async_copy(y_c.at[src_v.at[j + 1]], buf_b, sem_b)
            cp_a.wait()
            pltpu.sync_copy(buf_a, acc_sh.at[dst_v.at[j]], add=True)
            cp_b.wait()
            pltpu.sync_copy(buf_b, acc_sh.at[dst_v.at[j + 1]], add=True)
            return 0

        for g in range(CH_PER_TILE // IDXC):
            base = s * CH_PER_TILE + g * IDXC
            pltpu.sync_copy(src_hbm.at[pl.ds(base, IDXC)], src_v)
            pltpu.sync_copy(dst_hbm.at[p, pl.ds(base, IDXC)], dst_v)
            lax.fori_loop(0, IDXC // 2, _pair, 0)

        plsc.subcore_barrier()

        pltpu.sync_copy(acc_sh.at[pl.ds(rowbase, RPT)],
                        agg_hbm.at[c, p, pl.ds(rowbase, RPT)])
        plsc.subcore_barrier()


def _make_sc(with_deg):
    out_type = (jax.ShapeDtypeStruct((NC, NH, RH, HALF), jnp.float32),)
    scratch = [
        pltpu.VMEM((IDXC, CHUNK), jnp.int32),
        pltpu.VMEM((IDXC, CHUNK), jnp.int32),
        pltpu.VMEM((CHUNK, HALF), jnp.float32),
        pltpu.VMEM((CHUNK, HALF), jnp.float32),
    ]
    if with_deg:
        out_type = out_type + (
            jax.ShapeDtypeStruct((NH, RH, HALF), jnp.float32),)
    scratch.append(pltpu.VMEM_SHARED((RH, HALF), jnp.float32))
    scratch += [pltpu.SemaphoreType.DMA, pltpu.SemaphoreType.DMA]
    return pl.kernel(
        functools.partial(_sc_body, with_deg),
        out_type=out_type,
        mesh=_mesh,
        scratch_types=scratch,
        name="sage_sc_agg_deg" if with_deg else "sage_sc_agg",
    )


_sc_agg_deg = _make_sc(True)
_sc_agg = _make_sc(False)


def _tc_first_body(x_ref, wl_ref, wr_ref, bl_ref, y_ref, z_ref):
    xb = x_ref[...]
    yl = jnp.dot(xb, wl_ref[...], preferred_element_type=jnp.float32)
    y_ref[0] = yl[:, :HALF]
    y_ref[1] = yl[:, HALF:]
    z_ref[...] = (jnp.dot(xb, wr_ref[...], preferred_element_type=jnp.float32)
                  + bl_ref[...])


_tc_first = pl.pallas_call(
    _tc_first_body,
    grid=(GRID,),
    in_specs=[
        pl.BlockSpec((BR, D), lambda i: (i, 0)),
        pl.BlockSpec((D, D), lambda i: (0, 0)),
        pl.BlockSpec((D, D), lambda i: (0, 0)),
        pl.BlockSpec((1, D), lambda i: (0, 0)),
    ],
    out_specs=[
        pl.BlockSpec((NC, BR, HALF), lambda i: (0, i, 0)),
        pl.BlockSpec((BR, D), lambda i: (i, 0)),
    ],
    out_shape=[
        jax.ShapeDtypeStruct((NC, N, HALF), jnp.float32),
        jax.ShapeDtypeStruct((N, D), jnp.float32),
    ],
)


def _agg_spec():
    return pl.BlockSpec((NC, 1, BR, HALF), lambda i: (0, i // BPH, i % BPH, 0))


def _deg_spec():
    return pl.BlockSpec((1, BR, HALF), lambda i: (i // BPH, i % BPH, 0))


def _tc_mid_body(agg_ref, z_ref, deg_ref, wl_ref, wr_ref, bl_ref, y_ref, z2_ref):
    d = jnp.maximum(deg_ref[0][:, 0:1], 1.0)
    zb = z_ref[...]
    ha = jnp.maximum(agg_ref[0, 0] / d + zb[:, :HALF], 0.0)
    hb = jnp.maximum(agg_ref[1, 0] / d + zb[:, HALF:], 0.0)
    wl = wl_ref[...]
    wr = wr_ref[...]
    yl = (jnp.dot(ha, wl[:HALF, :], preferred_element_type=jnp.float32)
          + jnp.dot(hb, wl[HALF:, :], preferred_element_type=jnp.float32))
    y_ref[0] = yl[:, :HALF]
    y_ref[1] = yl[:, HALF:]
    z2_ref[...] = (jnp.dot(ha, wr[:HALF, :], preferred_element_type=jnp.float32)
                   + jnp.dot(hb, wr[HALF:, :], preferred_element_type=jnp.float32)
                   + bl_ref[...])


_tc_mid = pl.pallas_call(
    _tc_mid_body,
    grid=(GRID,),
    in_specs=[
        _agg_spec(),
        pl.BlockSpec((BR, D), lambda i: (i, 0)),
        _deg_spec(),
        pl.BlockSpec((D, D), lambda i: (0, 0)),
        pl.BlockSpec((D, D), lambda i: (0, 0)),
        pl.BlockSpec((1, D), lambda i: (0, 0)),
    ],
    out_specs=[
        pl.BlockSpec((NC, BR, HALF), lambda i: (0, i, 0)),
        pl.BlockSpec((BR, D), lambda i: (i, 0)),
    ],
    out_shape=[
        jax.ShapeDtypeStruct((NC, N, HALF), jnp.float32),
        jax.ShapeDtypeStruct((N, D), jnp.float32),
    ],
)


def _tc_final_body(agg_ref, z_ref, deg_ref, out_ref):
    d = jnp.maximum(deg_ref[0][:, 0:1], 1.0)
    zb = z_ref[...]
    out_ref[:, :HALF] = agg_ref[0, 0] / d + zb[:, :HALF]
    out_ref[:, HALF:] = agg_ref[1, 0] / d + zb[:, HALF:]


_tc_final = pl.pallas_call(
    _tc_final_body,
    grid=(GRID,),
    in_specs=[
        _agg_spec(),
        pl.BlockSpec((BR, D), lambda i: (i, 0)),
        _deg_spec(),
    ],
    out_specs=pl.BlockSpec((BR, D), lambda i: (i, 0)),
    out_shape=jax.ShapeDtypeStruct((N, D), jnp.float32),
)


def kernel(x, edge_index, Wl0, bl0, Wr0, Wl1, bl1, Wr1, Wl2, bl2, Wr2):
    src = edge_index[0].astype(jnp.int32)
    dst = edge_index[1].astype(jnp.int32)
    src2 = jnp.concatenate(
        [src, jnp.zeros((EP - E,), jnp.int32)]).reshape(NT, CHUNK)
    dstp = jnp.concatenate(
        [dst, jnp.full((EP - E,), N, jnp.int32)])
    dst_lo = jnp.where(dstp < NHN, dstp, DUMP)
    dst_hi = jnp.where(dstp >= NHN, dstp - NHN, DUMP)
    dst_hi = jnp.where(dst_hi > DUMP, DUMP, dst_hi)
    dst2 = jnp.stack([dst_lo.reshape(NT, CHUNK), dst_hi.reshape(NT, CHUNK)])

    y, z = _tc_first(x, Wl0, Wr0, bl0.reshape(1, D))
    agg, degw = _sc_agg_deg(y, src2, dst2)
    y, z = _tc_mid(agg, z, degw, Wl1, Wr1, bl1.reshape(1, D))
    (agg,) = _sc_agg(y, src2, dst2)
    y, z = _tc_mid(agg, z, degw, Wl2, Wr2, bl2.reshape(1, D))
    (agg,) = _sc_agg(y, src2, dst2)
    return _tc_final(agg, z, degw)


# speedup vs baseline: 1.9531x; 1.2246x over previous
import functools

import jax
import jax.numpy as jnp
from jax import lax
from jax.experimental import pallas as pl
from jax.experimental.pallas import tpu as pltpu
from jax.experimental.pallas import tpu_sc as plsc

N = 10000
E = 160000
D = 256
HALF = 128
NC = 2
NS = 16
NH = 2
NHN = N // NH
CHUNK = 128
CH_PER_TILE = 80
NT = NS * CH_PER_TILE
EP = NT * CHUNK
RPT = 320
RH = NS * RPT
DUMP = NHN
ZCH = -(-RPT // CHUNK)
IDXC = 8
NSLAB = CH_PER_TILE // IDXC

BR = 1000
GRID = N // BR
BPH = NHN // BR

_mesh = plsc.VectorSubcoreMesh(core_axis_name="c", subcore_axis_name="s")


def _sc_body(with_deg, *refs):
    if with_deg:
        (y_hbm, src_hbm, dst_hbm, agg_hbm, deg_hbm,
         src_v, dst_v, buf_a, buf_b, acc_sh,
         sem_ga, sem_gb, sem_sa, sem_sb, sem_i0, sem_i1) = refs
    else:
        (y_hbm, src_hbm, dst_hbm, agg_hbm,
         src_v, dst_v, buf_a, buf_b, acc_sh,
         sem_ga, sem_gb, sem_sa, sem_sb, sem_i0, sem_i1) = refs

    c = lax.axis_index("c")
    s = lax.axis_index("s")
    rowbase = s * RPT

    def _fill(buf, val):
        def _row(r, _):
            for k in range(HALF // 16):
                buf[r, pl.ds(k * 16, 16)] = jnp.full((16,), val, jnp.float32)
            return 0
        lax.fori_loop(0, CHUNK, _row, 0)

    def _zero_acc():
        for j in range(ZCH):
            n = min(CHUNK, RPT - j * CHUNK)
            pltpu.sync_copy(buf_b.at[pl.ds(0, n)],
                            acc_sh.at[pl.ds(rowbase + j * CHUNK, n)])

    def _stage(idx_src, m):
        slot = m % 2
        isem = sem_i0 if slot == 0 else sem_i1
        base = s * CH_PER_TILE + m * IDXC
        cs = pltpu.async_copy(src_hbm.at[pl.ds(base, IDXC)],
                              src_v.at[slot], isem)
        cd = pltpu.async_copy(idx_src.at[pl.ds(base, IDXC)],
                              dst_v.at[slot], isem)
        return (cs, cd)

    if with_deg:
        _fill(buf_a, 1.0)
        _fill(buf_b, 0.0)
        _zero_acc()
        plsc.subcore_barrier()

        dst_c = dst_hbm.at[c]
        pre = {0: _stage(dst_c, 0)}
        if NSLAB > 1:
            pre[1] = _stage(dst_c, 1)
        for m in range(NSLAB):
            pre[m][0].wait()
            pre[m][1].wait()
            slot = m % 2
            fired = [
                pltpu.async_copy(buf_a, acc_sh.at[dst_v.at[slot, r]],
                                 sem_sa, add=True)
                for r in range(IDXC)
            ]
            for d in fired:
                d.wait()
            if m + 2 < NSLAB:
                pre[m + 2] = _stage(dst_c, m + 2)

        plsc.subcore_barrier()
        pltpu.sync_copy(acc_sh.at[pl.ds(rowbase, RPT)],
                        deg_hbm.at[c, pl.ds(rowbase, RPT)])
        plsc.subcore_barrier()

    for p in range(NH):
        _fill(buf_b, 0.0)
        _zero_acc()
        plsc.subcore_barrier()

        y_c = y_hbm.at[c]
        dst_p = dst_hbm.at[p]
        NCHUNK = CH_PER_TILE

        bufs = (buf_a, buf_b)
        gsems = (sem_ga, sem_gb)
        ssems = (sem_sa, sem_sb)

        pre = {0: _stage(dst_p, 0)}
        if NSLAB > 1:
            pre[1] = _stage(dst_p, 1)
        pre[0][0].wait()
        pre[0][1].wait()

        gd = {}
        sd = {}
        for t in range(min(2, NCHUNK)):
            m, r = divmod(t, IDXC)
            gd[t] = pltpu.async_copy(y_c.at[src_v.at[m % 2, r]],
                                     bufs[t % 2], gsems[t % 2])

        for t in range(NCHUNK):
            b = t % 2
            m, r = divmod(t, IDXC)
            gd[t].wait()
            sd[t] = pltpu.async_copy(bufs[b], acc_sh.at[dst_v.at[m % 2, r]],
                                     ssems[b], add=True)
            nxt = t + 2
            if nxt < NCHUNK:
                mn, rn = divmod(nxt, IDXC)
                if rn == 0:
                    pre[mn][0].wait()
                    pre[mn][1].wait()
                sd[t].wait()
                gd[nxt] = pltpu.async_copy(y_c.at[src_v.at[mn % 2, rn]],
                                           bufs[b], gsems[b])
                if rn == 1 and mn + 1 < NSLAB:
                    pre[mn + 1] = _stage(dst_p, mn + 1)

        for t in (NCHUNK - 2, NCHUNK - 1):
            if t >= 0:
                sd[t].wait()

        plsc.subcore_barrier()

        pltpu.sync_copy(acc_sh.at[pl.ds(rowbase, RPT)],
                        agg_hbm.at[c, p, pl.ds(rowbase, RPT)])
        plsc.subcore_barrier()


def _make_sc(with_deg):
    out_type = (jax.ShapeDtypeStruct((NC, NH, RH, HALF), jnp.float32),)
    scratch = [
        pltpu.VMEM((2, IDXC, CHUNK), jnp.int32),
        pltpu.VMEM((2, IDXC, CHUNK), jnp.int32),
        pltpu.VMEM((CHUNK, HALF), jnp.float32),
        pltpu.VMEM((CHUNK, HALF), jnp.float32),
    ]
    if with_deg:
        out_type = out_type + (
            jax.ShapeDtypeStruct((NH, RH, HALF), jnp.float32),)
    scratch.append(pltpu.VMEM_SHARED((RH, HALF), jnp.float32))
    scratch += [pltpu.SemaphoreType.DMA] * 6
    return pl.kernel(
        functools.partial(_sc_body, with_deg),
        out_type=out_type,
        mesh=_mesh,
        scratch_types=scratch,
        name="sage_sc_agg_deg" if with_deg else "sage_sc_agg",
    )


_sc_agg_deg = _make_sc(True)
_sc_agg = _make_sc(False)


def _tc_first_body(x_ref, wl_ref, wr_ref, bl_ref, y_ref, z_ref):
    xb = x_ref[...]
    yl = jnp.dot(xb, wl_ref[...], preferred_element_type=jnp.float32)
    y_ref[0] = yl[:, :HALF]
    y_ref[1] = yl[:, HALF:]
    z_ref[...] = (jnp.dot(xb, wr_ref[...], preferred_element_type=jnp.float32)
                  + bl_ref[...])


_tc_first = pl.pallas_call(
    _tc_first_body,
    grid=(GRID,),
    in_specs=[
        pl.BlockSpec((BR, D), lambda i: (i, 0)),
        pl.BlockSpec((D, D), lambda i: (0, 0)),
        pl.BlockSpec((D, D), lambda i: (0, 0)),
        pl.BlockSpec((1, D), lambda i: (0, 0)),
    ],
    out_specs=[
        pl.BlockSpec((NC, BR, HALF), lambda i: (0, i, 0)),
        pl.BlockSpec((BR, D), lambda i: (i, 0)),
    ],
    out_shape=[
        jax.ShapeDtypeStruct((NC, N, HALF), jnp.float32),
        jax.ShapeDtypeStruct((N, D), jnp.float32),
    ],
)


def _agg_spec():
    return pl.BlockSpec((NC, 1, BR, HALF), lambda i: (0, i // BPH, i % BPH, 0))


def _deg_spec():
    return pl.BlockSpec((1, BR, HALF), lambda i: (i // BPH, i % BPH, 0))


def _tc_mid_body(agg_ref, z_ref, deg_ref, wl_ref, wr_ref, bl_ref, y_ref, z2_ref):
    d = jnp.maximum(deg_ref[0][:, 0:1], 1.0)
    zb = z_ref[...]
    ha = jnp.maximum(agg_ref[0, 0] / d + zb[:, :HALF], 0.0)
    hb = jnp.maximum(agg_ref[1, 0] / d + zb[:, HALF:], 0.0)
    wl = wl_ref[...]
    wr = wr_ref[...]
    yl = (jnp.dot(ha, wl[:HALF, :], preferred_element_type=jnp.float32)
          + jnp.dot(hb, wl[HALF:, :], preferred_element_type=jnp.float32))
    y_ref[0] = yl[:, :HALF]
    y_ref[1] = yl[:, HALF:]
    z2_ref[...] = (jnp.dot(ha, wr[:HALF, :], preferred_element_type=jnp.float32)
                   + jnp.dot(hb, wr[HALF:, :], preferred_element_type=jnp.float32)
                   + bl_ref[...])


_tc_mid = pl.pallas_call(
    _tc_mid_body,
    grid=(GRID,),
    in_specs=[
        _agg_spec(),
        pl.BlockSpec((BR, D), lambda i: (i, 0)),
        _deg_spec(),
        pl.BlockSpec((D, D), lambda i: (0, 0)),
        pl.BlockSpec((D, D), lambda i: (0, 0)),
        pl.BlockSpec((1, D), lambda i: (0, 0)),
    ],
    out_specs=[
        pl.BlockSpec((NC, BR, HALF), lambda i: (0, i, 0)),
        pl.BlockSpec((BR, D), lambda i: (i, 0)),
    ],
    out_shape=[
        jax.ShapeDtypeStruct((NC, N, HALF), jnp.float32),
        jax.ShapeDtypeStruct((N, D), jnp.float32),
    ],
)


def _tc_final_body(agg_ref, z_ref, deg_ref, out_ref):
    d = jnp.maximum(deg_ref[0][:, 0:1], 1.0)
    zb = z_ref[...]
    out_ref[:, :HALF] = agg_ref[0, 0] / d + zb[:, :HALF]
    out_ref[:, HALF:] = agg_ref[1, 0] / d + zb[:, HALF:]


_tc_final = pl.pallas_call(
    _tc_final_body,
    grid=(GRID,),
    in_specs=[
        _agg_spec(),
        pl.BlockSpec((BR, D), lambda i: (i, 0)),
        _deg_spec(),
    ],
    out_specs=pl.BlockSpec((BR, D), lambda i: (i, 0)),
    out_shape=jax.ShapeDtypeStruct((N, D), jnp.float32),
)


def kernel(x, edge_index, Wl0, bl0, Wr0, Wl1, bl1, Wr1, Wl2, bl2, Wr2):
    src = edge_index[0].astype(jnp.int32)
    dst = edge_index[1].astype(jnp.int32)
    src2 = jnp.concatenate(
        [src, jnp.zeros((EP - E,), jnp.int32)]).reshape(NT, CHUNK)
    dstp = jnp.concatenate(
        [dst, jnp.full((EP - E,), N, jnp.int32)])
    dst_lo = jnp.where(dstp < NHN, dstp, DUMP)
    dst_hi = jnp.where(dstp >= NHN, dstp - NHN, DUMP)
    dst_hi = jnp.where(dst_hi > DUMP, DUMP, dst_hi)
    dst2 = jnp.stack([dst_lo.reshape(NT, CHUNK), dst_hi.reshape(NT, CHUNK)])

    y, z = _tc_first(x, Wl0, Wr0, bl0.reshape(1, D))
    agg, degw = _sc_agg_deg(y, src2, dst2)
    y, z = _tc_mid(agg, z, degw, Wl1, Wr1, bl1.reshape(1, D))
    (agg,) = _sc_agg(y, src2, dst2)
    y, z = _tc_mid(agg, z, degw, Wl2, Wr2, bl2.reshape(1, D))
    (agg,) = _sc_agg(y, src2, dst2)
    return _tc_final(agg, z, degw)
